# baseline (speedup 1.0000x reference)
import functools

import numpy as np
import jax
import jax.numpy as jnp
from jax import lax
from jax.experimental import pallas as pl
from jax.experimental.pallas import tpu as pltpu
from jax.experimental.pallas import tpu_sc as plsc

_NC, _NS = 2, 16
_NW = _NC * _NS
_CHUNK = 128
_NBUF = 4

_BASE = 10000.0


def _cos_sin_tables(D, seq_end):
    inv_freq = 1.0 / (_BASE ** (np.arange(0, D, 2, dtype=np.float64) / D))
    t = np.arange(seq_end, dtype=np.float64)
    freqs = np.outer(t, inv_freq)
    freqs = np.concatenate((freqs, freqs), axis=-1)
    return (np.cos(freqs).astype(np.float32), np.sin(freqs).astype(np.float32))


_CT, _ST = _cos_sin_tables(16, 8)
_C64, _S64 = _cos_sin_tables(24, 64)


def _make_gather(N):
    assert N % (_NW * _CHUNK * _NBUF) == 0
    per_w = N // _NW
    n_chunks = per_w // _CHUNK
    n_outer = n_chunks // _NBUF
    mesh = plsc.VectorSubcoreMesh(core_axis_name="c", subcore_axis_name="s")
    f32 = jnp.float32

    @functools.partial(
        pl.kernel,
        mesh=mesh,
        compiler_params=pltpu.CompilerParams(use_tc_tiling_on_sc=False),
        out_type=[
            jax.ShapeDtypeStruct((N, 16), f32),
            jax.ShapeDtypeStruct((N, 16), f32),
            jax.ShapeDtypeStruct((N, 24), f32),
            jax.ShapeDtypeStruct((N, 24), f32),
            jax.ShapeDtypeStruct((N, 24), f32),
            jax.ShapeDtypeStruct((N, 24), f32),
        ],
        scratch_types=(
            [pltpu.VMEM((_CHUNK,), jnp.int32) for _ in range(3 * _NBUF)]
            + [
                buf
                for _ in range(_NBUF)
                for buf in (
                    pltpu.VMEM((_CHUNK, 16), f32),
                    pltpu.VMEM((_CHUNK, 16), f32),
                    pltpu.VMEM((_CHUNK, 24), f32),
                    pltpu.VMEM((_CHUNK, 24), f32),
                    pltpu.VMEM((_CHUNK, 24), f32),
                    pltpu.VMEM((_CHUNK, 24), f32),
                )
            ]
            + [pltpu.SemaphoreType.DMA for _ in range(3 * _NBUF)]
        ),
    )
    def gather_kernel(pt, py, px, ct, st, c64, s64,
                      o_ct, o_st, o_cy, o_sy, o_cx, o_sx, *scratch):
        idx = [scratch[3 * s:3 * s + 3] for s in range(_NBUF)]
        rows = [scratch[3 * _NBUF + 6 * s:3 * _NBUF + 6 * s + 6]
                for s in range(_NBUF)]
        sems = scratch[9 * _NBUF:]
        semi = sems[0:_NBUF]
        semg = sems[_NBUF:2 * _NBUF]
        semw = sems[2 * _NBUF:3 * _NBUF]
        outs = (o_ct, o_st, o_cy, o_sy, o_cx, o_sx)
        pos = (pt, py, px)
        tabs = (ct, st, c64, s64, c64, s64)

        wid = lax.axis_index("s") * _NC + lax.axis_index("c")
        base = wid * per_w

        def issue_idx(s, c):
            tok0 = base + c * _CHUNK
            for p, ib in zip(pos, idx[s]):
                pltpu.async_copy(p.at[pl.ds(tok0, _CHUNK)], ib, semi[s])

        def wait_idx(s):
            for p, ib in zip(pos, idx[s]):
                pltpu.make_async_copy(p.at[pl.ds(0, _CHUNK)], ib, semi[s]).wait()

        def issue_gathers(s):
            it, iy, ix = idx[s]
            for tab, iv, rb in zip(tabs, (it, it, iy, iy, ix, ix), rows[s]):
                pltpu.async_copy(tab.at[iv], rb, semg[s])

        def wait_gathers(s):
            it, iy, ix = idx[s]
            for tab, iv, rb in zip(tabs, (it, it, iy, iy, ix, ix), rows[s]):
                pltpu.make_async_copy(tab.at[iv], rb, semg[s]).wait()

        def issue_writes(s, c):
            tok0 = base + c * _CHUNK
            for rb, o in zip(rows[s], outs):
                pltpu.async_copy(rb, o.at[pl.ds(tok0, _CHUNK)], semw[s])

        def wait_writes(s):
            for rb, o in zip(rows[s], outs):
                pltpu.make_async_copy(rb, o.at[pl.ds(0, _CHUNK)], semw[s]).wait()

        for s in range(_NBUF):
            issue_idx(s, s)
        wait_idx(0)
        issue_gathers(0)

        def outer_body(g, carry):
            for k in range(_NBUF):
                s, s1 = k, (k + 1) % _NBUF
                i = g * _NBUF + k
                if k < _NBUF - 1:
                    @pl.when(g >= 1)
                    def _():
                        wait_writes(s1)
                    wait_idx(s1)
                    issue_gathers(s1)
                else:
                    @pl.when(g < n_outer - 1)
                    def _():
                        wait_writes(s1)
                        wait_idx(s1)
                        issue_gathers(s1)
                wait_gathers(s)
                @pl.when(g < n_outer - 1)
                def _():
                    issue_idx(s, i + _NBUF)
                issue_writes(s, i)
            return carry

        lax.fori_loop(0, n_outer, outer_body, 0)
        for s in range(_NBUF):
            wait_writes(s)

    return gather_kernel


def kernel(dim, pos_t, pos_y, pos_x, max_t, max_y, max_x):
    ntok, B = pos_t.shape
    N = ntok * B
    pt = pos_t.reshape(N).astype(jnp.int32)
    py = pos_y.reshape(N).astype(jnp.int32)
    px = pos_x.reshape(N).astype(jnp.int32)
    tabs = (jnp.asarray(_CT), jnp.asarray(_ST), jnp.asarray(_C64), jnp.asarray(_S64))
    o_ct, o_st, o_cy, o_sy, o_cx, o_sx = _make_gather(N)(pt, py, px, *tabs)
    shp16 = (ntok, B, 1, 16)
    shp24 = (ntok, B, 1, 24)
    return (o_ct.reshape(shp16), o_st.reshape(shp16),
            o_cy.reshape(shp24), o_sy.reshape(shp24),
            o_cx.reshape(shp24), o_sx.reshape(shp24))


# speedup vs baseline: 2.0188x; 2.0188x over previous
import functools

import numpy as np
import jax
import jax.numpy as jnp
from jax import lax
from jax.experimental import pallas as pl
from jax.experimental.pallas import tpu as pltpu
from jax.experimental.pallas import tpu_sc as plsc

_NC, _NS = 2, 16
_NW = _NC * _NS
_CHUNK = 128
_NBUF = 4

_BASE = 10000.0


def _cos_sin_tables(D, seq_end):
    inv_freq = 1.0 / (_BASE ** (np.arange(0, D, 2, dtype=np.float64) / D))
    t = np.arange(seq_end, dtype=np.float64)
    freqs = np.outer(t, inv_freq)
    freqs = np.concatenate((freqs, freqs), axis=-1)
    return (np.cos(freqs).astype(np.float32), np.sin(freqs).astype(np.float32))


_CT, _ST = _cos_sin_tables(16, 8)
_C64, _S64 = _cos_sin_tables(24, 64)


def _make_gather(N):
    assert N % (_NW * _CHUNK * _NBUF) == 0
    per_w = N // _NW
    n_chunks = per_w // _CHUNK
    n_outer = n_chunks // _NBUF
    mesh = plsc.VectorSubcoreMesh(core_axis_name="c", subcore_axis_name="s")
    f32 = jnp.float32

    @functools.partial(
        pl.kernel,
        mesh=mesh,
        compiler_params=pltpu.CompilerParams(use_tc_tiling_on_sc=False),
        out_type=[
            jax.ShapeDtypeStruct((N, 16), f32),
            jax.ShapeDtypeStruct((N, 16), f32),
            jax.ShapeDtypeStruct((N, 24), f32),
            jax.ShapeDtypeStruct((N, 24), f32),
            jax.ShapeDtypeStruct((N, 24), f32),
            jax.ShapeDtypeStruct((N, 24), f32),
        ],
        scratch_types=(
            [pltpu.VMEM((_CHUNK,), jnp.int32) for _ in range(3 * _NBUF)]
            + [
                buf
                for _ in range(_NBUF)
                for buf in (
                    pltpu.VMEM((_CHUNK, 16), f32),
                    pltpu.VMEM((_CHUNK, 16), f32),
                    pltpu.VMEM((_CHUNK, 24), f32),
                    pltpu.VMEM((_CHUNK, 24), f32),
                    pltpu.VMEM((_CHUNK, 24), f32),
                    pltpu.VMEM((_CHUNK, 24), f32),
                )
            ]
            + [
                pltpu.VMEM_SHARED((8, 16), f32),
                pltpu.VMEM_SHARED((8, 16), f32),
                pltpu.VMEM_SHARED((64, 24), f32),
                pltpu.VMEM_SHARED((64, 24), f32),
            ]
            + [pltpu.SemaphoreType.DMA for _ in range(3 * _NBUF)]
        ),
    )
    def gather_kernel(pt, py, px, ct_h, st_h, c64_h, s64_h,
                      o_ct, o_st, o_cy, o_sy, o_cx, o_sx, *scratch):
        idx = [scratch[3 * s:3 * s + 3] for s in range(_NBUF)]
        rows = [scratch[3 * _NBUF + 6 * s:3 * _NBUF + 6 * s + 6]
                for s in range(_NBUF)]
        ct, st, c64, s64 = scratch[9 * _NBUF:9 * _NBUF + 4]
        sems = scratch[9 * _NBUF + 4:]
        semi = sems[0:_NBUF]
        semg = sems[_NBUF:2 * _NBUF]
        semw = sems[2 * _NBUF:3 * _NBUF]
        outs = (o_ct, o_st, o_cy, o_sy, o_cx, o_sx)
        pos = (pt, py, px)
        tabs = (ct, st, c64, s64, c64, s64)

        wid = lax.axis_index("s") * _NC + lax.axis_index("c")
        base = wid * per_w

        def issue_idx(s, c):
            tok0 = base + c * _CHUNK
            for p, ib in zip(pos, idx[s]):
                pltpu.async_copy(p.at[pl.ds(tok0, _CHUNK)], ib, semi[s])

        def wait_idx(s):
            for p, ib in zip(pos, idx[s]):
                pltpu.make_async_copy(p.at[pl.ds(0, _CHUNK)], ib, semi[s]).wait()

        def issue_gathers(s):
            it, iy, ix = idx[s]
            for tab, iv, rb in zip(tabs, (it, it, iy, iy, ix, ix), rows[s]):
                pltpu.async_copy(tab.at[iv], rb, semg[s])

        def wait_gathers(s):
            it, iy, ix = idx[s]
            for tab, iv, rb in zip(tabs, (it, it, iy, iy, ix, ix), rows[s]):
                pltpu.make_async_copy(tab.at[iv], rb, semg[s]).wait()

        def issue_writes(s, c):
            tok0 = base + c * _CHUNK
            for rb, o in zip(rows[s], outs):
                pltpu.async_copy(rb, o.at[pl.ds(tok0, _CHUNK)], semw[s])

        def wait_writes(s):
            for rb, o in zip(rows[s], outs):
                pltpu.make_async_copy(rb, o.at[pl.ds(0, _CHUNK)], semw[s]).wait()

        for th, tv in zip((ct_h, st_h, c64_h, s64_h), (ct, st, c64, s64)):
            pltpu.sync_copy(th, tv)

        for s in range(_NBUF):
            issue_idx(s, s)
        wait_idx(0)
        issue_gathers(0)

        def outer_body(g, carry):
            for k in range(_NBUF):
                s, s1 = k, (k + 1) % _NBUF
                i = g * _NBUF + k
                if k < _NBUF - 1:
                    @pl.when(g >= 1)
                    def _():
                        wait_writes(s1)
                    wait_idx(s1)
                    issue_gathers(s1)
                else:
                    @pl.when(g < n_outer - 1)
                    def _():
                        wait_writes(s1)
                        wait_idx(s1)
                        issue_gathers(s1)
                wait_gathers(s)
                @pl.when(g < n_outer - 1)
                def _():
                    issue_idx(s, i + _NBUF)
                issue_writes(s, i)
            return carry

        lax.fori_loop(0, n_outer, outer_body, 0)
        for s in range(_NBUF):
            wait_writes(s)

    return gather_kernel


def kernel(dim, pos_t, pos_y, pos_x, max_t, max_y, max_x):
    ntok, B = pos_t.shape
    N = ntok * B
    pt = pos_t.reshape(N).astype(jnp.int32)
    py = pos_y.reshape(N).astype(jnp.int32)
    px = pos_x.reshape(N).astype(jnp.int32)
    tabs = (jnp.asarray(_CT), jnp.asarray(_ST), jnp.asarray(_C64), jnp.asarray(_S64))
    o_ct, o_st, o_cy, o_sy, o_cx, o_sx = _make_gather(N)(pt, py, px, *tabs)
    shp16 = (ntok, B, 1, 16)
    shp24 = (ntok, B, 1, 24)
    return (o_ct.reshape(shp16), o_st.reshape(shp16),
            o_cy.reshape(shp24), o_sy.reshape(shp24),
            o_cx.reshape(shp24), o_sx.reshape(shp24))
